# baseline (speedup 1.0000x reference)
import functools

import jax
import jax.numpy as jnp
from jax import lax
from jax.experimental import pallas as pl
from jax.experimental.pallas import tpu as pltpu
from jax.experimental.pallas import tpu_sc as plsc

VOCAB = 100000
WORD_DIM = 300
HIDDEN = 128
B = 4096
L = 200


_MM_BLK = 5000


def _proj_body(wv_ref, wt_ref, out_ref):
    out_ref[...] = lax.dot_general(
        wv_ref[...], wt_ref[...],
        dimension_numbers=(((1,), (0,)), ((), ())),
        preferred_element_type=jnp.float32,
    )


def _project_table(word_vectors, W_proj):
    return pl.pallas_call(
        _proj_body,
        grid=(VOCAB // _MM_BLK,),
        in_specs=[
            pl.BlockSpec((_MM_BLK, WORD_DIM), lambda i: (i, 0)),
            pl.BlockSpec((WORD_DIM, HIDDEN), lambda i: (0, 0)),
        ],
        out_specs=pl.BlockSpec((_MM_BLK, HIDDEN), lambda i: (i, 0)),
        out_shape=jax.ShapeDtypeStruct((VOCAB, HIDDEN), jnp.float32),
    )(word_vectors, W_proj.T)


_INFO = plsc.get_sparse_core_info()
_NC, _NS = _INFO.num_cores, _INFO.num_subcores
_NW = _NC * _NS
_TOKENS = B * L
_PER_W = _TOKENS // _NW
_CHUNK = 128
_NCHUNK = _PER_W // _CHUNK
_NBUF = 4
_NSUP = _NCHUNK // _NBUF


def _gather_body(table_hbm, idx_hbm, out_hbm, idx_v,
                 r0, r1, r2, r3, g0, g1, g2, g3, w0, w1, w2, w3):
    rows = [r0, r1, r2, r3]
    gsem = [g0, g1, g2, g3]
    wsem = [w0, w1, w2, w3]
    wid = lax.axis_index("s") * _NC + lax.axis_index("c")
    base = wid * _PER_W
    pltpu.sync_copy(idx_hbm.at[wid], idx_v)

    for b in range(_NBUF):
        pltpu.async_copy(table_hbm.at[idx_v.at[b]], rows[b], gsem[b])

    def sup(g, carry):
        for b in range(_NBUF):
            j = g * _NBUF + b
            pltpu.make_async_copy(
                table_hbm.at[idx_v.at[0]], rows[b], gsem[b]).wait()
            pltpu.async_copy(
                rows[b], out_hbm.at[pl.ds(base + j * _CHUNK, _CHUNK)], wsem[b])
        for b in range(_NBUF):
            nxt = g * _NBUF + b + _NBUF
            pltpu.make_async_copy(
                rows[b], out_hbm.at[pl.ds(base, _CHUNK)], wsem[b]).wait()

            @pl.when(nxt < _NCHUNK)
            def _():
                pltpu.async_copy(table_hbm.at[idx_v.at[nxt]], rows[b], gsem[b])
        return carry

    lax.fori_loop(0, _NSUP, sup, 0)


def _gather_rows(table, idx):
    mesh = plsc.VectorSubcoreMesh(core_axis_name="c", subcore_axis_name="s")
    k = functools.partial(
        pl.kernel,
        mesh=mesh,
        out_type=jax.ShapeDtypeStruct((_TOKENS, HIDDEN), jnp.float32),
        scratch_types=[
            pltpu.VMEM((_NCHUNK, _CHUNK), jnp.int32),
        ] + [pltpu.VMEM((_CHUNK, HIDDEN), jnp.float32)] * _NBUF
          + [pltpu.SemaphoreType.DMA] * (2 * _NBUF),
    )(_gather_body)
    return k(table, idx.reshape(_NW, _NCHUNK, _CHUNK))


def kernel(x, word_vectors, W_proj):
    table = _project_table(word_vectors, W_proj)
    idx = x.reshape(_TOKENS).astype(jnp.int32)
    out = _gather_rows(table, idx)
    return out.reshape(B, L, HIDDEN)
